# baseline (speedup 1.0000x reference)
import functools

import jax
import jax.numpy as jnp
from jax import lax
from jax.experimental import pallas as pl
from jax.experimental.pallas import tpu as pltpu
from jax.experimental.pallas import tpu_sc as plsc


def _proj_body(ns_ref, wa_ref, wb_ref, pa_ref, pb_ref):
    x = ns_ref[...]
    pa_ref[...] = jnp.dot(
        x, wa_ref[...], preferred_element_type=jnp.float32).astype(jnp.bfloat16)
    pb_ref[...] = jnp.dot(
        x, wb_ref[...], preferred_element_type=jnp.float32).astype(jnp.bfloat16)


def _project_nodes(ns, wa, wb, block=2000):
    n, s = ns.shape
    f = wa.shape[1]
    grid = n // block
    return pl.pallas_call(
        _proj_body,
        grid=(grid,),
        in_specs=[
            pl.BlockSpec((block, s), lambda i: (i, 0)),
            pl.BlockSpec((s, f), lambda i: (0, 0)),
            pl.BlockSpec((s, f), lambda i: (0, 0)),
        ],
        out_specs=[
            pl.BlockSpec((block, f), lambda i: (i, 0)),
            pl.BlockSpec((block, f), lambda i: (i, 0)),
        ],
        out_shape=[
            jax.ShapeDtypeStruct((n, f), jnp.bfloat16),
            jax.ShapeDtypeStruct((n, f), jnp.bfloat16),
        ],
    )(ns, wa, wb)


_CH = 128


def _make_gather_sum(e_total, f):
    info = plsc.get_sparse_core_info()
    nc, ns_ = info.num_cores, info.num_subcores
    nw = nc * ns_
    assert e_total % nw == 0
    cpw = e_total // nw
    nfull = cpw // _CH
    tail = cpw - nfull * _CH
    assert nfull % 2 == 1
    half = e_total // 2
    assert half % cpw == 0

    mesh = plsc.VectorSubcoreMesh(core_axis_name="c", subcore_axis_name="s")

    buf_set = [
        pltpu.VMEM((_CH,), jnp.int32),
        pltpu.VMEM((_CH,), jnp.int32),
        pltpu.VMEM((_CH, f), jnp.bfloat16),
        pltpu.VMEM((_CH, f), jnp.bfloat16),
        pltpu.SemaphoreType.DMA,
    ]
    scratch = buf_set + buf_set
    if tail:
        scratch += [
            pltpu.VMEM((tail,), jnp.int32),
            pltpu.VMEM((tail,), jnp.int32),
            pltpu.VMEM((tail, f), jnp.bfloat16),
            pltpu.VMEM((tail, f), jnp.bfloat16),
        ]

    @functools.partial(
        pl.kernel,
        mesh=mesh,
        out_type=jax.ShapeDtypeStruct((half, 2 * f), jnp.bfloat16),
        scratch_types=scratch,
        compiler_params=pltpu.CompilerParams(use_tc_tiling_on_sc=False),
    )
    def gather_sum(tsrc, tdst, srci, dsti, out, *bufs):
        wid = lax.axis_index("s") * nc + lax.axis_index("c")
        base = wid * cpw
        h = base // half
        col = h * f
        sets = (bufs[0:5], bufs[5:10])

        def fire(ci, s):
            i_s, i_d, r_a, r_b, sem = s
            cbase = base + ci * _CH
            pltpu.sync_copy(srci.at[pl.ds(cbase, _CH)], i_s)
            pltpu.sync_copy(dsti.at[pl.ds(cbase, _CH)], i_d)
            pltpu.make_async_copy(tsrc.at[i_s], r_a, sem).start()
            pltpu.make_async_copy(tdst.at[i_d], r_b, sem).start()

        def process(ci, ch, s):
            i_s, i_d, r_a, r_b, sem = s
            pltpu.make_async_copy(tsrc.at[i_s], r_a, sem).wait()
            pltpu.make_async_copy(tdst.at[i_d], r_b, sem).wait()

            def add_row(r, carry):
                for c in range(f // 32):
                    sl = (r, pl.ds(32 * c, 32))
                    r_a[sl] = r_a[sl] + r_b[sl]
                return carry

            lax.fori_loop(0, ch, add_row, 0)
            pltpu.sync_copy(r_a, out.at[pl.ds(base + ci * _CH - h * half, ch),
                                        pl.ds(col, f)])

        npairs = (nfull - 1) // 2
        fire(0, sets[0])

        def body(k, carry):
            fire(2 * k + 1, sets[1])
            process(2 * k, _CH, sets[0])
            fire(2 * k + 2, sets[0])
            process(2 * k + 1, _CH, sets[1])
            return carry

        lax.fori_loop(0, npairs, body, 0)
        process(nfull - 1, _CH, sets[0])
        if tail:
            i_st, i_dt, r_at, r_bt = bufs[10:14]
            cbase = base + nfull * _CH
            sem = sets[1][4]
            pltpu.sync_copy(srci.at[pl.ds(cbase, tail)], i_st)
            pltpu.sync_copy(dsti.at[pl.ds(cbase, tail)], i_dt)
            pltpu.make_async_copy(tsrc.at[i_st], r_at, sem).start()
            pltpu.make_async_copy(tdst.at[i_dt], r_bt, sem).start()
            pltpu.make_async_copy(tsrc.at[i_st], r_at, sem).wait()
            pltpu.make_async_copy(tdst.at[i_dt], r_bt, sem).wait()

            def add_row_t(r, carry):
                for c in range(f // 32):
                    sl = (r, pl.ds(32 * c, 32))
                    r_at[sl] = r_at[sl] + r_bt[sl]
                return carry

            lax.fori_loop(0, tail, add_row_t, 0)
            pltpu.sync_copy(r_at, out.at[pl.ds(cbase - h * half, tail),
                                         pl.ds(col, f)])

    return gather_sum


def _edge_body(g_ref, eft_ref, dt_ref, eye_ref, w1c_ref, w1d_ref, b1_ref,
               w2_ref, b2_ref, gam_ref, bet_ref, *rest):
    o_ref = rest[-1]
    dn0 = (((0,), (0,)), ((), ()))
    eft = eft_ref[...]
    gt = lax.dot_general(eye_ref[0], g_ref[...], (((1,), (1,)), ((), ())),
                         preferred_element_type=jnp.float32)
    pre = (gt
           + lax.dot_general(w1c_ref[...], eft, dn0,
                             preferred_element_type=jnp.float32)
           + lax.dot_general(w1d_ref[...], dt_ref[...], dn0,
                             preferred_element_type=jnp.float32)
           + b1_ref[...])
    h = pre * jax.nn.sigmoid(pre)
    pre2 = lax.dot_general(w2_ref[...], h, dn0,
                           preferred_element_type=jnp.float32) + b2_ref[...]
    h2 = pre2 * jax.nn.sigmoid(pre2)
    y = eft + h2
    mu = jnp.mean(y, axis=0, keepdims=True)
    var = jnp.mean((y - mu) * (y - mu), axis=0, keepdims=True)
    o_ref[...] = (y - mu) * lax.rsqrt(var + 1e-5) * gam_ref[...] + bet_ref[...]


def _edge_mlp_range(gsum2_k, ef_t, d_t, w1c, w1d, b1c, w2, b2c, gamc, betc,
                    blk0, out_prev, block=3200):
    f, e = ef_t.shape
    r = d_t.shape[0]
    hb = (gsum2_k.shape[0]) // block
    eyes = jnp.stack([
        jnp.concatenate([jnp.eye(f, dtype=jnp.bfloat16),
                         jnp.zeros((f, f), jnp.bfloat16)], axis=1),
        jnp.concatenate([jnp.zeros((f, f), jnp.bfloat16),
                         jnp.eye(f, dtype=jnp.bfloat16)], axis=1),
    ])
    full = lambda a, b: pl.BlockSpec((a, b), lambda i, h: (0, 0))
    col = lambda i, h: (0, blk0 + i + h * hb)
    in_specs = [
        pl.BlockSpec((block, 2 * f), lambda i, h: (i, 0)),
        pl.BlockSpec((f, block), col),
        pl.BlockSpec((r, block), col),
        pl.BlockSpec((1, f, 2 * f), lambda i, h: (h, 0, 0)),
        full(f, f),
        full(r, f),
        full(f, 1),
        full(f, f),
        full(f, 1),
        full(f, 1),
        full(f, 1),
    ]
    args = [gsum2_k, ef_t, d_t, eyes, w1c, w1d, b1c, w2, b2c, gamc, betc]
    aliases = {}
    if out_prev is not None:
        in_specs.append(pl.BlockSpec(memory_space=pl.ANY))
        args.append(out_prev)
        aliases = {11: 0}
    return pl.pallas_call(
        _edge_body,
        grid=(hb, 2),
        in_specs=in_specs,
        out_specs=pl.BlockSpec((f, block), col),
        out_shape=jax.ShapeDtypeStruct((f, e), jnp.float32),
        input_output_aliases=aliases,
    )(*args)


def kernel(node_scalars, edge_feats, d, W1, b1, W2, b2, ln_gamma, ln_beta, edge_index):
    n, s = node_scalars.shape
    e, f = edge_feats.shape

    w1a = W1[:s]
    w1b = W1[s:2 * s]
    w1c = W1[2 * s:2 * s + f]
    w1d = W1[2 * s + f:]

    tsrc, tdst = _project_nodes(node_scalars, w1a, w1b)

    src = edge_index[0].astype(jnp.int32)
    dst = edge_index[1].astype(jnp.int32)

    K = 5
    block = 3200
    ec = e // K
    gather = _make_gather_sum(ec, f)
    ef_t = edge_feats.T
    d_t = d.T
    b1c = b1.reshape(f, 1)
    b2c = b2.reshape(f, 1)
    gamc = ln_gamma.reshape(f, 1)
    betc = ln_beta.reshape(f, 1)

    gsums = [
        gather(tsrc, tdst, src[k * ec:(k + 1) * ec], dst[k * ec:(k + 1) * ec])
        for k in range(K)
    ]
    out_t = None
    for k in range(K):
        out_t = _edge_mlp_range(
            gsums[k], ef_t, d_t, w1c, w1d, b1c, W2, b2c, gamc, betc,
            blk0=k * (ec // block), out_prev=out_t, block=block,
        )
    return out_t.T


# speedup vs baseline: 1.4515x; 1.4515x over previous
import functools

import jax
import jax.numpy as jnp
from jax import lax
from jax.experimental import pallas as pl
from jax.experimental.pallas import tpu as pltpu
from jax.experimental.pallas import tpu_sc as plsc


def _proj_body(ns_ref, wa_ref, wb_ref, pa_ref, pb_ref):
    x = ns_ref[...]
    pa_ref[...] = jnp.dot(x, wa_ref[...], preferred_element_type=jnp.float32)
    pb_ref[...] = jnp.dot(x, wb_ref[...], preferred_element_type=jnp.float32)


def _project_nodes(ns, wa, wb, block=2000):
    n, s = ns.shape
    f = wa.shape[1]
    grid = n // block
    return pl.pallas_call(
        _proj_body,
        grid=(grid,),
        in_specs=[
            pl.BlockSpec((block, s), lambda i: (i, 0)),
            pl.BlockSpec((s, f), lambda i: (0, 0)),
            pl.BlockSpec((s, f), lambda i: (0, 0)),
        ],
        out_specs=[
            pl.BlockSpec((block, f), lambda i: (i, 0)),
            pl.BlockSpec((block, f), lambda i: (i, 0)),
        ],
        out_shape=[
            jax.ShapeDtypeStruct((n, f), jnp.float32),
            jax.ShapeDtypeStruct((n, f), jnp.float32),
        ],
    )(ns, wa, wb)


_CH = 128


def _make_gather_sum(e_total, f):
    info = plsc.get_sparse_core_info()
    nc, ns_ = info.num_cores, info.num_subcores
    nw = nc * ns_
    assert e_total % nw == 0
    cpw = e_total // nw
    nfull = cpw // _CH
    tail = cpw - nfull * _CH
    assert nfull % 2 == 1
    half = e_total // 2
    assert half % cpw == 0

    mesh = plsc.VectorSubcoreMesh(core_axis_name="c", subcore_axis_name="s")

    buf_set = [
        pltpu.VMEM((_CH,), jnp.int32),
        pltpu.VMEM((_CH,), jnp.int32),
        pltpu.VMEM((_CH, f), jnp.float32),
        pltpu.VMEM((_CH, f), jnp.float32),
        pltpu.SemaphoreType.DMA,
    ]
    scratch = buf_set + buf_set
    if tail:
        scratch += [
            pltpu.VMEM((tail,), jnp.int32),
            pltpu.VMEM((tail,), jnp.int32),
            pltpu.VMEM((tail, f), jnp.float32),
            pltpu.VMEM((tail, f), jnp.float32),
        ]

    @functools.partial(
        pl.kernel,
        mesh=mesh,
        out_type=jax.ShapeDtypeStruct((half, 2 * f), jnp.float32),
        scratch_types=scratch,
        compiler_params=pltpu.CompilerParams(use_tc_tiling_on_sc=False),
    )
    def gather_sum(tsrc, tdst, srci, dsti, out, *bufs):
        wid = lax.axis_index("s") * nc + lax.axis_index("c")
        base = wid * cpw
        h = base // half
        col = h * f
        sets = (bufs[0:5], bufs[5:10])

        def fire(ci, s):
            i_s, i_d, r_a, r_b, sem = s
            cbase = base + ci * _CH
            pltpu.sync_copy(srci.at[pl.ds(cbase, _CH)], i_s)
            pltpu.sync_copy(dsti.at[pl.ds(cbase, _CH)], i_d)
            pltpu.make_async_copy(tsrc.at[i_s], r_a, sem).start()
            pltpu.make_async_copy(tdst.at[i_d], r_b, sem).start()

        def process(ci, ch, s):
            i_s, i_d, r_a, r_b, sem = s
            pltpu.make_async_copy(tsrc.at[i_s], r_a, sem).wait()
            pltpu.make_async_copy(tdst.at[i_d], r_b, sem).wait()

            def add_row(r, carry):
                for c in range(f // 16):
                    sl = (r, pl.ds(16 * c, 16))
                    plsc.addupdate(r_a.at[sl], r_b[sl])
                return carry

            lax.fori_loop(0, ch, add_row, 0)
            pltpu.sync_copy(r_a, out.at[pl.ds(base + ci * _CH - h * half, ch),
                                        pl.ds(col, f)])

        npairs = (nfull - 1) // 2
        fire(0, sets[0])

        def body(k, carry):
            fire(2 * k + 1, sets[1])
            process(2 * k, _CH, sets[0])
            fire(2 * k + 2, sets[0])
            process(2 * k + 1, _CH, sets[1])
            return carry

        lax.fori_loop(0, npairs, body, 0)
        process(nfull - 1, _CH, sets[0])
        if tail:
            i_st, i_dt, r_at, r_bt = bufs[10:14]
            cbase = base + nfull * _CH
            sem = sets[1][4]
            pltpu.sync_copy(srci.at[pl.ds(cbase, tail)], i_st)
            pltpu.sync_copy(dsti.at[pl.ds(cbase, tail)], i_dt)
            pltpu.make_async_copy(tsrc.at[i_st], r_at, sem).start()
            pltpu.make_async_copy(tdst.at[i_dt], r_bt, sem).start()
            pltpu.make_async_copy(tsrc.at[i_st], r_at, sem).wait()
            pltpu.make_async_copy(tdst.at[i_dt], r_bt, sem).wait()

            def add_row_t(r, carry):
                for c in range(f // 16):
                    sl = (r, pl.ds(16 * c, 16))
                    plsc.addupdate(r_at.at[sl], r_bt[sl])
                return carry

            lax.fori_loop(0, tail, add_row_t, 0)
            pltpu.sync_copy(r_at, out.at[pl.ds(cbase - h * half, tail),
                                         pl.ds(col, f)])

    return gather_sum


def _edge_body(g_ref, eft_ref, dt_ref, eye_ref, w1c_ref, w1d_ref, b1_ref,
               w2_ref, b2_ref, gam_ref, bet_ref, *rest):
    o_ref = rest[-1]
    dn0 = (((0,), (0,)), ((), ()))
    eft = eft_ref[...]
    gt = lax.dot_general(eye_ref[0], g_ref[...], (((1,), (1,)), ((), ())),
                         preferred_element_type=jnp.float32)
    pre = (gt
           + lax.dot_general(w1c_ref[...], eft, dn0,
                             preferred_element_type=jnp.float32)
           + lax.dot_general(w1d_ref[...], dt_ref[...], dn0,
                             preferred_element_type=jnp.float32)
           + b1_ref[...])
    h = pre * jax.nn.sigmoid(pre)
    pre2 = lax.dot_general(w2_ref[...], h, dn0,
                           preferred_element_type=jnp.float32) + b2_ref[...]
    h2 = pre2 * jax.nn.sigmoid(pre2)
    y = eft + h2
    mu = jnp.mean(y, axis=0, keepdims=True)
    var = jnp.mean((y - mu) * (y - mu), axis=0, keepdims=True)
    o_ref[...] = (y - mu) * lax.rsqrt(var + 1e-5) * gam_ref[...] + bet_ref[...]


def _edge_mlp_range(gsum2_k, ef_t, d_t, w1c, w1d, b1c, w2, b2c, gamc, betc,
                    blk0, out_prev, block=3200):
    f, e = ef_t.shape
    r = d_t.shape[0]
    hb = (gsum2_k.shape[0]) // block
    eyes = jnp.stack([
        jnp.concatenate([jnp.eye(f, dtype=jnp.float32),
                         jnp.zeros((f, f), jnp.float32)], axis=1),
        jnp.concatenate([jnp.zeros((f, f), jnp.float32),
                         jnp.eye(f, dtype=jnp.float32)], axis=1),
    ])
    full = lambda a, b: pl.BlockSpec((a, b), lambda i, h: (0, 0))
    col = lambda i, h: (0, blk0 + i + h * hb)
    in_specs = [
        pl.BlockSpec((block, 2 * f), lambda i, h: (i, 0)),
        pl.BlockSpec((f, block), col),
        pl.BlockSpec((r, block), col),
        pl.BlockSpec((1, f, 2 * f), lambda i, h: (h, 0, 0)),
        full(f, f),
        full(r, f),
        full(f, 1),
        full(f, f),
        full(f, 1),
        full(f, 1),
        full(f, 1),
    ]
    args = [gsum2_k, ef_t, d_t, eyes, w1c, w1d, b1c, w2, b2c, gamc, betc]
    aliases = {}
    if out_prev is not None:
        in_specs.append(pl.BlockSpec(memory_space=pl.ANY))
        args.append(out_prev)
        aliases = {11: 0}
    return pl.pallas_call(
        _edge_body,
        grid=(hb, 2),
        in_specs=in_specs,
        out_specs=pl.BlockSpec((f, block), col),
        out_shape=jax.ShapeDtypeStruct((f, e), jnp.float32),
        input_output_aliases=aliases,
    )(*args)


def kernel(node_scalars, edge_feats, d, W1, b1, W2, b2, ln_gamma, ln_beta, edge_index):
    n, s = node_scalars.shape
    e, f = edge_feats.shape

    w1a = W1[:s]
    w1b = W1[s:2 * s]
    w1c = W1[2 * s:2 * s + f]
    w1d = W1[2 * s + f:]

    tsrc, tdst = _project_nodes(node_scalars, w1a, w1b)

    src = edge_index[0].astype(jnp.int32)
    dst = edge_index[1].astype(jnp.int32)

    K = 25
    block = 3200
    ec = e // K
    gather = _make_gather_sum(ec, f)
    ef_t = edge_feats.T
    d_t = d.T
    b1c = b1.reshape(f, 1)
    b2c = b2.reshape(f, 1)
    gamc = ln_gamma.reshape(f, 1)
    betc = ln_beta.reshape(f, 1)

    gsums = [
        gather(tsrc, tdst, src[k * ec:(k + 1) * ec], dst[k * ec:(k + 1) * ec])
        for k in range(K)
    ]
    out_t = None
    for k in range(K):
        out_t = _edge_mlp_range(
            gsums[k], ef_t, d_t, w1c, w1d, b1c, W2, b2c, gamc, betc,
            blk0=k * (ec // block), out_prev=out_t, block=block,
        )
    return out_t.T


# speedup vs baseline: 1.5747x; 1.0849x over previous
import functools

import jax
import jax.numpy as jnp
from jax import lax
from jax.experimental import pallas as pl
from jax.experimental.pallas import tpu as pltpu
from jax.experimental.pallas import tpu_sc as plsc


def _proj_body(ns_ref, wa_ref, wb_ref, pa_ref, pb_ref):
    x = ns_ref[...]
    pa_ref[...] = jnp.dot(x, wa_ref[...], preferred_element_type=jnp.float32)
    pb_ref[...] = jnp.dot(x, wb_ref[...], preferred_element_type=jnp.float32)


def _project_nodes(ns, wa, wb, block=2000):
    n, s = ns.shape
    f = wa.shape[1]
    grid = n // block
    return pl.pallas_call(
        _proj_body,
        grid=(grid,),
        in_specs=[
            pl.BlockSpec((block, s), lambda i: (i, 0)),
            pl.BlockSpec((s, f), lambda i: (0, 0)),
            pl.BlockSpec((s, f), lambda i: (0, 0)),
        ],
        out_specs=[
            pl.BlockSpec((block, f), lambda i: (i, 0)),
            pl.BlockSpec((block, f), lambda i: (i, 0)),
        ],
        out_shape=[
            jax.ShapeDtypeStruct((n, f), jnp.float32),
            jax.ShapeDtypeStruct((n, f), jnp.float32),
        ],
    )(ns, wa, wb)


_CH = 128


def _make_gather_sum(e_total, f):
    info = plsc.get_sparse_core_info()
    nc, ns_ = info.num_cores, info.num_subcores
    nw = nc * ns_
    assert e_total % nw == 0
    cpw = e_total // nw
    nfull = cpw // _CH
    tail = cpw - nfull * _CH
    assert nfull % 2 == 1
    half = e_total // 2
    assert half % cpw == 0

    mesh = plsc.VectorSubcoreMesh(core_axis_name="c", subcore_axis_name="s")

    buf_set = [
        pltpu.VMEM((_CH,), jnp.int32),
        pltpu.VMEM((_CH,), jnp.int32),
        pltpu.VMEM((_CH, f), jnp.float32),
        pltpu.VMEM((_CH, f), jnp.float32),
        pltpu.SemaphoreType.DMA,
    ]
    scratch = buf_set + buf_set
    if tail:
        scratch += [
            pltpu.VMEM((tail,), jnp.int32),
            pltpu.VMEM((tail,), jnp.int32),
            pltpu.VMEM((tail, f), jnp.float32),
            pltpu.VMEM((tail, f), jnp.float32),
        ]

    @functools.partial(
        pl.kernel,
        mesh=mesh,
        out_type=jax.ShapeDtypeStruct((half, 2 * f), jnp.float32),
        scratch_types=scratch,
        compiler_params=pltpu.CompilerParams(use_tc_tiling_on_sc=False),
    )
    def gather_sum(tsrc, tdst, srci, dsti, out, *bufs):
        wid = lax.axis_index("s") * nc + lax.axis_index("c")
        base = wid * cpw
        h = base // half
        col = h * f
        sets = (bufs[0:5], bufs[5:10])

        def fire(ci, s):
            i_s, i_d, r_a, r_b, sem = s
            cbase = base + ci * _CH
            pltpu.sync_copy(srci.at[pl.ds(cbase, _CH)], i_s)
            pltpu.sync_copy(dsti.at[pl.ds(cbase, _CH)], i_d)
            pltpu.make_async_copy(tsrc.at[i_s], r_a, sem).start()
            pltpu.make_async_copy(tdst.at[i_d], r_b, sem).start()

        def process(ci, ch, s):
            i_s, i_d, r_a, r_b, sem = s
            pltpu.make_async_copy(tsrc.at[i_s], r_a, sem).wait()
            pltpu.make_async_copy(tdst.at[i_d], r_b, sem).wait()

            def add_row(r, carry):
                for c in range(f // 16):
                    sl = (r, pl.ds(16 * c, 16))
                    plsc.addupdate(r_a.at[sl], r_b[sl])
                return carry

            lax.fori_loop(0, ch, add_row, 0)
            pltpu.sync_copy(r_a, out.at[pl.ds(base + ci * _CH - h * half, ch),
                                        pl.ds(col, f)])

        npairs = (nfull - 1) // 2
        fire(0, sets[0])

        def body(k, carry):
            fire(2 * k + 1, sets[1])
            process(2 * k, _CH, sets[0])
            fire(2 * k + 2, sets[0])
            process(2 * k + 1, _CH, sets[1])
            return carry

        lax.fori_loop(0, npairs, body, 0)
        process(nfull - 1, _CH, sets[0])
        if tail:
            i_st, i_dt, r_at, r_bt = bufs[10:14]
            cbase = base + nfull * _CH
            sem = sets[1][4]
            pltpu.sync_copy(srci.at[pl.ds(cbase, tail)], i_st)
            pltpu.sync_copy(dsti.at[pl.ds(cbase, tail)], i_dt)
            pltpu.make_async_copy(tsrc.at[i_st], r_at, sem).start()
            pltpu.make_async_copy(tdst.at[i_dt], r_bt, sem).start()
            pltpu.make_async_copy(tsrc.at[i_st], r_at, sem).wait()
            pltpu.make_async_copy(tdst.at[i_dt], r_bt, sem).wait()

            def add_row_t(r, carry):
                for c in range(f // 16):
                    sl = (r, pl.ds(16 * c, 16))
                    plsc.addupdate(r_at.at[sl], r_bt[sl])
                return carry

            lax.fori_loop(0, tail, add_row_t, 0)
            pltpu.sync_copy(r_at, out.at[pl.ds(cbase - h * half, tail),
                                         pl.ds(col, f)])

    return gather_sum


def _edge_body(g_ref, eft_ref, dt_ref, eye_ref, w1c_ref, w1d_ref, b1_ref,
               w2_ref, b2_ref, gam_ref, bet_ref, *rest):
    o_ref = rest[-1]
    dn0 = (((0,), (0,)), ((), ()))
    eft = eft_ref[...]
    gt = lax.dot_general(eye_ref[0], g_ref[...], (((1,), (1,)), ((), ())),
                         preferred_element_type=jnp.float32)
    pre = (gt
           + lax.dot_general(w1c_ref[...], eft, dn0,
                             preferred_element_type=jnp.float32)
           + lax.dot_general(w1d_ref[...], dt_ref[...], dn0,
                             preferred_element_type=jnp.float32)
           + b1_ref[...])
    h = pre * jax.nn.sigmoid(pre)
    pre2 = lax.dot_general(w2_ref[...], h, dn0,
                           preferred_element_type=jnp.float32) + b2_ref[...]
    h2 = pre2 * jax.nn.sigmoid(pre2)
    y = eft + h2
    mu = jnp.mean(y, axis=0, keepdims=True)
    var = jnp.mean((y - mu) * (y - mu), axis=0, keepdims=True)
    o_ref[...] = (y - mu) * lax.rsqrt(var + 1e-5) * gam_ref[...] + bet_ref[...]


def _edge_mlp_range(gsum2_k, ef_t, d_t, w1c, w1d, b1c, w2, b2c, gamc, betc,
                    blk0, out_prev, block=3200):
    f, e = ef_t.shape
    r = d_t.shape[0]
    hb = (gsum2_k.shape[0]) // block
    eyes = jnp.stack([
        jnp.concatenate([jnp.eye(f, dtype=jnp.float32),
                         jnp.zeros((f, f), jnp.float32)], axis=1),
        jnp.concatenate([jnp.zeros((f, f), jnp.float32),
                         jnp.eye(f, dtype=jnp.float32)], axis=1),
    ])
    full = lambda a, b: pl.BlockSpec((a, b), lambda i, h: (0, 0))
    col = lambda i, h: (0, blk0 + i + h * hb)
    in_specs = [
        pl.BlockSpec((block, 2 * f), lambda i, h: (i, 0)),
        pl.BlockSpec((f, block), col),
        pl.BlockSpec((r, block), col),
        pl.BlockSpec((1, f, 2 * f), lambda i, h: (h, 0, 0)),
        full(f, f),
        full(r, f),
        full(f, 1),
        full(f, f),
        full(f, 1),
        full(f, 1),
        full(f, 1),
    ]
    args = [gsum2_k, ef_t, d_t, eyes, w1c, w1d, b1c, w2, b2c, gamc, betc]
    aliases = {}
    if out_prev is not None:
        in_specs.append(pl.BlockSpec(memory_space=pl.ANY))
        args.append(out_prev)
        aliases = {11: 0}
    return pl.pallas_call(
        _edge_body,
        grid=(hb, 2),
        in_specs=in_specs,
        out_specs=pl.BlockSpec((f, block), col),
        out_shape=jax.ShapeDtypeStruct((f, e), jnp.float32),
        input_output_aliases=aliases,
    )(*args)


def kernel(node_scalars, edge_feats, d, W1, b1, W2, b2, ln_gamma, ln_beta, edge_index):
    n, s = node_scalars.shape
    e, f = edge_feats.shape

    w1a = W1[:s]
    w1b = W1[s:2 * s]
    w1c = W1[2 * s:2 * s + f]
    w1d = W1[2 * s + f:]

    tsrc, tdst = _project_nodes(node_scalars, w1a, w1b)

    src = edge_index[0].astype(jnp.int32)
    dst = edge_index[1].astype(jnp.int32)

    K = 5
    block = 3200
    ec = e // K
    gather = _make_gather_sum(ec, f)
    ef_t = edge_feats.T
    d_t = d.T
    b1c = b1.reshape(f, 1)
    b2c = b2.reshape(f, 1)
    gamc = ln_gamma.reshape(f, 1)
    betc = ln_beta.reshape(f, 1)

    gsums = [
        gather(tsrc, tdst, src[k * ec:(k + 1) * ec], dst[k * ec:(k + 1) * ec])
        for k in range(K)
    ]
    out_t = None
    for k in range(K):
        out_t = _edge_mlp_range(
            gsums[k], ef_t, d_t, w1c, w1d, b1c, W2, b2c, gamc, betc,
            blk0=k * (ec // block), out_prev=out_t, block=block,
        )
    return out_t.T


# speedup vs baseline: 1.5933x; 1.0119x over previous
import functools

import jax
import jax.numpy as jnp
from jax import lax
from jax.experimental import pallas as pl
from jax.experimental.pallas import tpu as pltpu
from jax.experimental.pallas import tpu_sc as plsc


def _proj_body(nsa_ref, nsb_ref, wa_ref, wb_ref, pa_ref, pb_ref):
    xa = nsa_ref[...]
    xb = nsb_ref[...]
    wa = wa_ref[...]
    wb = wb_ref[...]
    pa_ref[...] = jnp.concatenate(
        [jnp.dot(xa, wa, preferred_element_type=jnp.float32),
         jnp.dot(xb, wa, preferred_element_type=jnp.float32)], axis=1)
    pb_ref[...] = jnp.concatenate(
        [jnp.dot(xa, wb, preferred_element_type=jnp.float32),
         jnp.dot(xb, wb, preferred_element_type=jnp.float32)], axis=1)


def _project_nodes(ns, wa, wb, block=5000):
    n, s = ns.shape
    f = wa.shape[1]
    grid = (n // 2) // block
    pa2, pb2 = pl.pallas_call(
        _proj_body,
        grid=(grid,),
        in_specs=[
            pl.BlockSpec((block, s), lambda i: (i, 0)),
            pl.BlockSpec((block, s), lambda i, g=grid: (i + g, 0)),
            pl.BlockSpec((s, f), lambda i: (0, 0)),
            pl.BlockSpec((s, f), lambda i: (0, 0)),
        ],
        out_specs=[
            pl.BlockSpec((block, 2 * f), lambda i: (i, 0)),
            pl.BlockSpec((block, 2 * f), lambda i: (i, 0)),
        ],
        out_shape=[
            jax.ShapeDtypeStruct((n // 2, 2 * f), jnp.float32),
            jax.ShapeDtypeStruct((n // 2, 2 * f), jnp.float32),
        ],
    )(ns, ns, wa, wb)
    return pa2.reshape(n, f), pb2.reshape(n, f)


_CH = 128


def _make_gather_sum(e_total, f):
    info = plsc.get_sparse_core_info()
    nc, ns_ = info.num_cores, info.num_subcores
    nw = nc * ns_
    assert e_total % nw == 0
    cpw = e_total // nw
    nfull = cpw // _CH
    tail = cpw - nfull * _CH
    assert nfull % 2 == 1
    half = e_total // 2
    assert half % cpw == 0

    mesh = plsc.VectorSubcoreMesh(core_axis_name="c", subcore_axis_name="s")

    buf_set = [
        pltpu.VMEM((_CH,), jnp.int32),
        pltpu.VMEM((_CH,), jnp.int32),
        pltpu.VMEM((_CH, f), jnp.float32),
        pltpu.VMEM((_CH, f), jnp.float32),
        pltpu.SemaphoreType.DMA,
    ]
    scratch = buf_set + buf_set
    if tail:
        scratch += [
            pltpu.VMEM((tail,), jnp.int32),
            pltpu.VMEM((tail,), jnp.int32),
            pltpu.VMEM((tail, f), jnp.float32),
            pltpu.VMEM((tail, f), jnp.float32),
        ]

    @functools.partial(
        pl.kernel,
        mesh=mesh,
        out_type=jax.ShapeDtypeStruct((half, 2 * f), jnp.float32),
        scratch_types=scratch,
        compiler_params=pltpu.CompilerParams(use_tc_tiling_on_sc=False),
    )
    def gather_sum(tsrc, tdst, srci, dsti, out, *bufs):
        wid = lax.axis_index("s") * nc + lax.axis_index("c")
        base = wid * cpw
        h = base // half
        col = h * f
        sets = (bufs[0:5], bufs[5:10])

        def fire(ci, s):
            i_s, i_d, r_a, r_b, sem = s
            cbase = base + ci * _CH
            pltpu.sync_copy(srci.at[pl.ds(cbase, _CH)], i_s)
            pltpu.sync_copy(dsti.at[pl.ds(cbase, _CH)], i_d)
            pltpu.make_async_copy(tsrc.at[i_s], r_a, sem).start()
            pltpu.make_async_copy(tdst.at[i_d], r_b, sem).start()

        def process(ci, ch, s):
            i_s, i_d, r_a, r_b, sem = s
            pltpu.make_async_copy(tsrc.at[i_s], r_a, sem).wait()
            pltpu.make_async_copy(tdst.at[i_d], r_b, sem).wait()

            def add_row(r, carry):
                for c in range(f // 16):
                    sl = (r, pl.ds(16 * c, 16))
                    plsc.addupdate(r_a.at[sl], r_b[sl])
                return carry

            lax.fori_loop(0, ch, add_row, 0)
            pltpu.sync_copy(r_a, out.at[pl.ds(base + ci * _CH - h * half, ch),
                                        pl.ds(col, f)])

        npairs = (nfull - 1) // 2
        fire(0, sets[0])

        def body(k, carry):
            fire(2 * k + 1, sets[1])
            process(2 * k, _CH, sets[0])
            fire(2 * k + 2, sets[0])
            process(2 * k + 1, _CH, sets[1])
            return carry

        lax.fori_loop(0, npairs, body, 0)
        process(nfull - 1, _CH, sets[0])
        if tail:
            i_st, i_dt, r_at, r_bt = bufs[10:14]
            cbase = base + nfull * _CH
            sem = sets[1][4]
            pltpu.sync_copy(srci.at[pl.ds(cbase, tail)], i_st)
            pltpu.sync_copy(dsti.at[pl.ds(cbase, tail)], i_dt)
            pltpu.make_async_copy(tsrc.at[i_st], r_at, sem).start()
            pltpu.make_async_copy(tdst.at[i_dt], r_bt, sem).start()
            pltpu.make_async_copy(tsrc.at[i_st], r_at, sem).wait()
            pltpu.make_async_copy(tdst.at[i_dt], r_bt, sem).wait()

            def add_row_t(r, carry):
                for c in range(f // 16):
                    sl = (r, pl.ds(16 * c, 16))
                    plsc.addupdate(r_at.at[sl], r_bt[sl])
                return carry

            lax.fori_loop(0, tail, add_row_t, 0)
            pltpu.sync_copy(r_at, out.at[pl.ds(cbase - h * half, tail),
                                         pl.ds(col, f)])

    return gather_sum


def _edge_body(g_ref, eft_ref, dt_ref, eye_ref, w1c_ref, w1d_ref, b1_ref,
               w2_ref, b2_ref, gam_ref, bet_ref, *rest):
    o_ref = rest[-1]
    dn0 = (((0,), (0,)), ((), ()))
    eft = eft_ref[...]
    gt = lax.dot_general(eye_ref[0], g_ref[...], (((1,), (1,)), ((), ())),
                         preferred_element_type=jnp.float32)
    pre = (gt
           + lax.dot_general(w1c_ref[...], eft, dn0,
                             preferred_element_type=jnp.float32)
           + lax.dot_general(w1d_ref[...], dt_ref[...], dn0,
                             preferred_element_type=jnp.float32)
           + b1_ref[...])
    h = pre * jax.nn.sigmoid(pre)
    pre2 = lax.dot_general(w2_ref[...], h, dn0,
                           preferred_element_type=jnp.float32) + b2_ref[...]
    h2 = pre2 * jax.nn.sigmoid(pre2)
    y = eft + h2
    mu = jnp.mean(y, axis=0, keepdims=True)
    var = jnp.mean((y - mu) * (y - mu), axis=0, keepdims=True)
    o_ref[...] = (y - mu) * lax.rsqrt(var + 1e-5) * gam_ref[...] + bet_ref[...]


def _edge_mlp_range(gsum2_k, ef_t, d_t, w1c, w1d, b1c, w2, b2c, gamc, betc,
                    blk0, out_prev, block=3200):
    f, e = ef_t.shape
    r = d_t.shape[0]
    hb = (gsum2_k.shape[0]) // block
    eyes = jnp.stack([
        jnp.concatenate([jnp.eye(f, dtype=jnp.float32),
                         jnp.zeros((f, f), jnp.float32)], axis=1),
        jnp.concatenate([jnp.zeros((f, f), jnp.float32),
                         jnp.eye(f, dtype=jnp.float32)], axis=1),
    ])
    full = lambda a, b: pl.BlockSpec((a, b), lambda i, h: (0, 0))
    col = lambda i, h: (0, blk0 + i + h * hb)
    in_specs = [
        pl.BlockSpec((block, 2 * f), lambda i, h: (i, 0)),
        pl.BlockSpec((f, block), col),
        pl.BlockSpec((r, block), col),
        pl.BlockSpec((1, f, 2 * f), lambda i, h: (h, 0, 0)),
        full(f, f),
        full(r, f),
        full(f, 1),
        full(f, f),
        full(f, 1),
        full(f, 1),
        full(f, 1),
    ]
    args = [gsum2_k, ef_t, d_t, eyes, w1c, w1d, b1c, w2, b2c, gamc, betc]
    aliases = {}
    if out_prev is not None:
        in_specs.append(pl.BlockSpec(memory_space=pl.ANY))
        args.append(out_prev)
        aliases = {11: 0}
    return pl.pallas_call(
        _edge_body,
        grid=(hb, 2),
        in_specs=in_specs,
        out_specs=pl.BlockSpec((f, block), col),
        out_shape=jax.ShapeDtypeStruct((f, e), jnp.float32),
        input_output_aliases=aliases,
    )(*args)


def kernel(node_scalars, edge_feats, d, W1, b1, W2, b2, ln_gamma, ln_beta, edge_index):
    n, s = node_scalars.shape
    e, f = edge_feats.shape

    w1a = W1[:s]
    w1b = W1[s:2 * s]
    w1c = W1[2 * s:2 * s + f]
    w1d = W1[2 * s + f:]

    tsrc, tdst = _project_nodes(node_scalars, w1a, w1b)

    src = edge_index[0].astype(jnp.int32)
    dst = edge_index[1].astype(jnp.int32)
    nh = n // 2
    src = jnp.where(src < nh, src * 2, (src - nh) * 2 + 1)
    dst = jnp.where(dst < nh, dst * 2, (dst - nh) * 2 + 1)

    K = 5
    block = 3200
    ec = e // K
    gather = _make_gather_sum(ec, f)
    ef_t = edge_feats.T
    d_t = d.T
    b1c = b1.reshape(f, 1)
    b2c = b2.reshape(f, 1)
    gamc = ln_gamma.reshape(f, 1)
    betc = ln_beta.reshape(f, 1)

    gsums = [
        gather(tsrc, tdst, src[k * ec:(k + 1) * ec], dst[k * ec:(k + 1) * ec])
        for k in range(K)
    ]
    out_t = None
    for k in range(K):
        out_t = _edge_mlp_range(
            gsums[k], ef_t, d_t, w1c, w1d, b1c, W2, b2c, gamc, betc,
            blk0=k * (ec // block), out_prev=out_t, block=block,
        )
    return out_t.T
